# Initial kernel scaffold; baseline (speedup 1.0000x reference)
import jax
import jax.numpy as jnp
from jax.experimental import pallas as pl


def kernel(x, y):
    raise NotImplementedError("write your pallas kernel here")


# speedup vs baseline: 2.0737x; 2.0737x over previous
import functools

import numpy as np
import jax
import jax.numpy as jnp
from jax.experimental import pallas as pl

_NUM_PROJS = 128


def _sobol2_np(n):
    bits = 30
    ms = [1]
    for k in range(1, bits):
        ms.append((2 * ms[k - 1]) ^ ms[k - 1])
    v0 = [1 << (bits - 1 - k) for k in range(bits)]
    v1 = [ms[k] << (bits - 1 - k) for k in range(bits)]
    x0, x1 = 0, 0
    out = np.zeros((n, 2), dtype=np.float64)
    for i in range(1, n):
        c = 0
        j = i - 1
        while j & 1:
            j >>= 1
            c += 1
        x0 ^= v0[c]
        x1 ^= v1[c]
        out[i, 0] = x0 / float(1 << bits)
        out[i, 1] = x1 / float(1 << bits)
    return out


def _theta_np(num_projs):
    net = _sobol2_np(num_projs)
    alpha = net[:, 0:1]
    tau = net[:, 1:2]
    r = 2.0 * np.sqrt(np.maximum(tau - tau ** 2, 0.0))
    theta = np.concatenate([
        r * np.cos(2.0 * np.pi * alpha),
        r * np.sin(2.0 * np.pi * alpha),
        1.0 - 2.0 * tau,
    ], axis=1)
    return theta.astype(np.float32)


_THETA = _theta_np(_NUM_PROJS)


def _bitonic_sort_axis0(a):
    n, c = a.shape
    k = 2
    while k <= n:
        j = k // 2
        while j >= 1:
            g = n // (2 * j)
            b = a.reshape(g, 2, j, c)
            lo = jnp.minimum(b[:, 0], b[:, 1])
            hi = jnp.maximum(b[:, 0], b[:, 1])
            if k < n:
                gi = jax.lax.broadcasted_iota(jnp.int32, (g, 1, 1), 0)
                asc = ((gi * (2 * j)) & k) == 0
                first = jnp.where(asc, lo, hi)
                second = jnp.where(asc, hi, lo)
            else:
                first, second = lo, hi
            a = jnp.stack([first, second], axis=1).reshape(n, c)
            j //= 2
        k *= 2
    return a


def _tc_body(x_ref, y_ref, theta_ref, o_ref):
    x = x_ref[0]
    y = y_ref[0]
    th = theta_ref[...]
    xp = (x[:, 0:1] * th[0:1, :] + x[:, 1:2] * th[1:2, :]
          + x[:, 2:3] * th[2:3, :])
    yp = (y[:, 0:1] * th[0:1, :] + y[:, 1:2] * th[1:2, :]
          + y[:, 2:3] * th[2:3, :])
    a = jnp.concatenate([xp, yp], axis=1)
    a = _bitonic_sort_axis0(a)
    p = xp.shape[1]
    d = a[:, :p] - a[:, p:]
    o_ref[0, 0, :] = jnp.sum(d * d, axis=0)


@functools.partial(jax.jit, static_argnames=("interpret",))
def _swd_tc(x, y, theta_t, interpret=False):
    b, n, _ = x.shape
    p = theta_t.shape[1]
    s = pl.pallas_call(
        _tc_body,
        grid=(b,),
        in_specs=[
            pl.BlockSpec((1, n, 3), lambda i: (i, 0, 0)),
            pl.BlockSpec((1, n, 3), lambda i: (i, 0, 0)),
            pl.BlockSpec((3, p), lambda i: (0, 0)),
        ],
        out_specs=pl.BlockSpec((1, 1, p), lambda i: (i, 0, 0)),
        out_shape=jax.ShapeDtypeStruct((b, 1, p), jnp.float32),
        interpret=interpret,
    )(x, y, theta_t)
    return jnp.mean(jnp.sqrt(jnp.mean(s[:, 0, :], axis=1)))


def kernel(x, y):
    theta_t = jnp.asarray(_THETA.T)
    return _swd_tc(x, y, theta_t)
